# Initial kernel scaffold; baseline (speedup 1.0000x reference)
import jax
import jax.numpy as jnp
from jax.experimental import pallas as pl


def kernel(x, I):
    raise NotImplementedError("write your pallas kernel here")


# speedup vs baseline: 16.2300x; 16.2300x over previous
import functools

import jax
import jax.numpy as jnp
from jax import lax
from jax.experimental import pallas as pl
from jax.experimental.pallas import tpu as pltpu
from jax.experimental.pallas import tpu_sc as plsc

_GAMMA = 0.01
_INV_GAMMA = 100.0

_B, _G, _S, _L = 64, 2048, 64, 4
_NBH = 2
_NGT = 16
_BL = _B // _NBH
_GC = _G // _NGT
_NGQ = _GC // 16
_BB = 8
_NBB = _BL // _BB


def _sc_clause(x_hbm, idx_hbm, outm_hbm, outs_hbm, xloc, idxv, outm_v, outs_v):
    wid = lax.axis_index("s") * 2 + lax.axis_index("c")
    bh = wid // _NGT
    gt = wid % _NGT

    pltpu.sync_copy(x_hbm.at[pl.ds(bh * (_BL * _G), _BL * _G)], xloc)
    pltpu.sync_copy(idx_hbm.at[gt], idxv)

    for gq in range(_NGQ):
        c0 = gq * 16

        def bblk_body(bblk, _, c0=c0):
            b0 = bblk * _BB

            def s_body(s, st):
                i0 = idxv[s, 0, pl.ds(c0, 16)]
                i1 = idxv[s, 1, pl.ds(c0, 16)]
                i2 = idxv[s, 2, pl.ds(c0, 16)]
                i3 = idxv[s, 3, pl.ds(c0, 16)]
                new_m, new_s = [], []
                for j in range(_BB):
                    boff = jnp.full((16,), (b0 + j) * _G, jnp.int32)
                    a = plsc.load_gather(xloc, [boff + i0])
                    a = a * plsc.load_gather(xloc, [boff + i1])
                    a = a * plsc.load_gather(xloc, [boff + i2])
                    a = a * plsc.load_gather(xloc, [boff + i3])
                    m_old, s_old = st[j], st[_BB + j]
                    m_new = jnp.maximum(m_old, a)
                    s_new = (s_old * jnp.exp((m_old - m_new) * _INV_GAMMA)
                             + jnp.exp((a - m_new) * _INV_GAMMA))
                    new_m.append(m_new)
                    new_s.append(s_new)
                return tuple(new_m) + tuple(new_s)

            init = ((jnp.full((16,), -1e30, jnp.float32),) * _BB
                    + (jnp.zeros((16,), jnp.float32),) * _BB)
            fin = lax.fori_loop(0, _S, s_body, init)
            for j in range(_BB):
                outm_v[b0 + j, pl.ds(c0, 16)] = fin[j]
                outs_v[b0 + j, pl.ds(c0, 16)] = fin[_BB + j]
            return 0

        lax.fori_loop(0, _NBB, bblk_body, 0)

    pltpu.sync_copy(outm_v, outm_hbm.at[bh, gt])
    pltpu.sync_copy(outs_v, outs_hbm.at[bh, gt])


_sc_call = functools.partial(
    pl.kernel,
    out_type=[
        jax.ShapeDtypeStruct((_NBH, _NGT, _BL, _GC), jnp.float32),
        jax.ShapeDtypeStruct((_NBH, _NGT, _BL, _GC), jnp.float32),
    ],
    mesh=plsc.VectorSubcoreMesh(core_axis_name="c", subcore_axis_name="s"),
    compiler_params=pltpu.CompilerParams(needs_layout_passes=False),
    scratch_types=[
        pltpu.VMEM((_BL * _G,), jnp.float32),
        pltpu.VMEM((_S, _L, _GC), jnp.int32),
        pltpu.VMEM((_BL, _GC), jnp.float32),
        pltpu.VMEM((_BL, _GC), jnp.float32),
    ],
)(_sc_clause)


def _fin_body(m_ref, s_ref, o_ref):
    o_ref[...] = m_ref[...] + _GAMMA * jnp.log(s_ref[...])


_finish = pl.pallas_call(
    _fin_body,
    out_shape=jax.ShapeDtypeStruct((_B, _G), jnp.float32),
)


def kernel(x, I):
    idx = jnp.transpose(I[0], (1, 2, 0))
    idx = idx.reshape(_S, _L, _NGT, _GC).transpose(2, 0, 1, 3)
    outm4, outs4 = _sc_call(x.reshape(-1), idx)
    m = jnp.transpose(outm4, (0, 2, 1, 3)).reshape(_B, _G)
    sv = jnp.transpose(outs4, (0, 2, 1, 3)).reshape(_B, _G)
    return _finish(m, sv)
